# baseline (speedup 1.0000x reference)
import functools

import jax
import jax.numpy as jnp
from jax import lax
from jax.experimental import pallas as pl
from jax.experimental.pallas import tpu as pltpu
from jax.experimental.pallas import tpu_sc as plsc

_N = 10000
_E = 320000
_L = 128
_OUT = 3

_BE = 2000
_BN = 2000

_NC = 2
_NS = 16
_NW = _NC * _NS

_f32 = jnp.float32


def _dot(a, b):
    return jnp.dot(a.astype(jnp.bfloat16), b.astype(jnp.bfloat16),
                   preferred_element_type=_f32)


def _ln_apply(p, x):
    mu = x.mean(-1, keepdims=True)
    var = x.var(-1, keepdims=True)
    return (x - mu) / jnp.sqrt(var + 1e-5) * p["g"] + p["b"]


def _mlp3_body(x_ref, w1, b1, w2, b2, w3, b3, o_ref):
    h = jnp.maximum(_dot(x_ref[...], w1[...]) + b1[...], 0.0)
    h = jnp.maximum(_dot(h, w2[...]) + b2[...], 0.0)
    o_ref[...] = _dot(h, w3[...]) + b3[...]


def _cat3_mlp3_body(gs, gr, e_ref, w1, b1, w2, b2, w3, b3, o_ref):
    e_in = jnp.concatenate([gs[...], gr[...], e_ref[...]], axis=-1)
    h = jnp.maximum(_dot(e_in, w1[...]) + b1[...], 0.0)
    h = jnp.maximum(_dot(h, w2[...]) + b2[...], 0.0)
    o_ref[...] = _dot(h, w3[...]) + b3[...]


def _cat2_mlp3_body(n_ref, a_ref, w1, b1, w2, b2, w3, b3, o_ref):
    n_in = jnp.concatenate([n_ref[...], a_ref[...]], axis=-1)
    h = jnp.maximum(_dot(n_in, w1[...]) + b1[...], 0.0)
    h = jnp.maximum(_dot(h, w2[...]) + b2[...], 0.0)
    o_ref[...] = _dot(h, w3[...]) + b3[...]


def _full(a):
    return pl.BlockSpec(a.shape, lambda i: (0, 0))


def _mlp_weights(mlp):
    out = []
    for p in mlp:
        out.append(p["W"])
        out.append(p["b"].reshape(1, -1))
    return out


def _mlp3_pallas(x, mlp, dout, bs):
    rows, din = x.shape
    ws = _mlp_weights(mlp)
    return pl.pallas_call(
        _mlp3_body,
        grid=(rows // bs,),
        in_specs=[pl.BlockSpec((bs, din), lambda i: (i, 0))] + [_full(w) for w in ws],
        out_specs=pl.BlockSpec((bs, dout), lambda i: (i, 0)),
        out_shape=jax.ShapeDtypeStruct((rows, dout), _f32),
    )(x, *ws)


def _edge_mlp(gs, gr, edges, mlp):
    ws = _mlp_weights(mlp)
    bspec = pl.BlockSpec((_BE, _L), lambda i: (i, 0))
    return pl.pallas_call(
        _cat3_mlp3_body,
        grid=(_E // _BE,),
        in_specs=[bspec, bspec, bspec] + [_full(w) for w in ws],
        out_specs=bspec,
        out_shape=jax.ShapeDtypeStruct((_E, _L), _f32),
    )(gs, gr, edges, *ws)


def _node_mlp(nodes, agg, mlp):
    ws = _mlp_weights(mlp)
    bspec = pl.BlockSpec((_BN, _L), lambda i: (i, 0))
    return pl.pallas_call(
        _cat2_mlp3_body,
        grid=(_N // _BN,),
        in_specs=[bspec, bspec] + [_full(w) for w in ws],
        out_specs=bspec,
        out_shape=jax.ShapeDtypeStruct((_N, _L), _f32),
    )(nodes, agg, *ws)


_EPAD = 327680
_EPWP = _EPAD // _NW
_GK = 128
_NGRP = _EPWP // _GK


def _sc_gather(table, senders_pad, receivers_pad):
    mesh = plsc.VectorSubcoreMesh(core_axis_name="c", subcore_axis_name="s")

    @functools.partial(
        pl.kernel,
        out_type=[jax.ShapeDtypeStruct((_EPAD, _L), _f32)] * 2,
        mesh=mesh,
        scratch_types=[
            pltpu.VMEM((_EPWP,), jnp.int32),
            pltpu.VMEM((_EPWP,), jnp.int32),
            [pltpu.VMEM((_GK, _L), _f32) for _ in range(2)],
            [pltpu.VMEM((_GK, _L), _f32) for _ in range(2)],
            [pltpu.SemaphoreType.DMA for _ in range(2)],
            [pltpu.SemaphoreType.DMA for _ in range(2)],
        ],
    )
    def k(t_hbm, s_hbm, r_hbm, gs_hbm, gr_hbm,
          sidx, ridx, srows, rrows, gsem, wsem):
        wid = lax.axis_index("s") * _NC + lax.axis_index("c")
        base = wid * _EPWP
        pltpu.sync_copy(s_hbm.at[pl.ds(base, _EPWP)], sidx)
        pltpu.sync_copy(r_hbm.at[pl.ds(base, _EPWP)], ridx)

        def fire(j, b):
            off = j * _GK
            pltpu.async_copy(t_hbm.at[sidx.at[pl.ds(off, _GK)]],
                             srows[b], gsem[b])
            pltpu.async_copy(t_hbm.at[ridx.at[pl.ds(off, _GK)]],
                             rrows[b], gsem[b])

        def drain_g(b):
            pltpu.make_async_copy(t_hbm.at[pl.ds(0, _GK)], srows[b],
                                  gsem[b]).wait()
            pltpu.make_async_copy(t_hbm.at[pl.ds(0, _GK)], rrows[b],
                                  gsem[b]).wait()

        def write(j, b):
            dst = pl.ds(base + j * _GK, _GK)
            pltpu.async_copy(srows[b], gs_hbm.at[dst], wsem[b])
            pltpu.async_copy(rrows[b], gr_hbm.at[dst], wsem[b])

        def drain_w(b):
            pltpu.make_async_copy(srows[b], gs_hbm.at[pl.ds(0, _GK)],
                                  wsem[b]).wait()
            pltpu.make_async_copy(rrows[b], gr_hbm.at[pl.ds(0, _GK)],
                                  wsem[b]).wait()

        fire(0, 0)
        fire(1, 1)

        def body(i, carry):
            for b in range(2):
                j = 2 * i + b
                drain_g(b)
                write(j, b)
                drain_w(b)

                @pl.when(i < _NGRP // 2 - 1)
                def _():
                    fire(j + 2, b)
            return carry

        lax.fori_loop(0, _NGRP // 2, body, 0)

    return k(table, senders_pad, receivers_pad)


def _gnn_step(carry, blk):
    nodes, edges, s_pad, r_pad, receivers = carry
    gs, gr = _sc_gather(nodes, s_pad, r_pad)
    h3 = _edge_mlp(gs, gr, edges, blk["edge_mlp"])
    e_upd = _ln_apply(blk["edge_ln"], h3)
    agg = jax.ops.segment_sum(e_upd, receivers, num_segments=_N)
    u3 = _node_mlp(nodes, agg, blk["node_mlp"])
    nodes = nodes + _ln_apply(blk["node_ln"], u3)
    edges = edges + e_upd
    return (nodes, edges, s_pad, r_pad, receivers), None


def kernel(x, edge_index, edge_features, params):
    senders = edge_index[0].astype(jnp.int32)
    receivers = edge_index[1].astype(jnp.int32)
    pad = jnp.zeros((_EPAD - _E,), jnp.int32)
    s_pad = jnp.concatenate([senders, pad])
    r_pad = jnp.concatenate([receivers, pad])

    nodes = _ln_apply(params["node_enc"]["ln"],
                      _mlp3_pallas(x, params["node_enc"]["mlp"], _L, _BN))
    edges = _ln_apply(params["edge_enc"]["ln"],
                      _mlp3_pallas(edge_features, params["edge_enc"]["mlp"],
                                   _L, _BE))

    stacked = jax.tree.map(lambda *xs: jnp.stack(xs), *params["gnn"])
    (nodes, edges, _, _, _), _ = lax.scan(
        _gnn_step, (nodes, edges, s_pad, r_pad, receivers), stacked)

    return _mlp3_pallas(nodes, params["dec"]["mlp"], _OUT, _BN)
